# baseline (speedup 1.0000x reference)
import functools

import jax
import jax.numpy as jnp
from jax import lax
from jax.experimental import pallas as pl
from jax.experimental.pallas import tpu as pltpu


def _gemm_body(x_ref, wt_ref, out_ref):
    @pl.when(pl.program_id(1) == 0)
    def _zero():
        out_ref[...] = jnp.zeros_like(out_ref)

    out_ref[...] += jnp.dot(x_ref[...], wt_ref[...],
                            preferred_element_type=jnp.float32)


def _gate_body(xg_ref, out_ref, keys_ref, *, k):
    xg = xg_ref[0]
    bits = lax.bitcast_convert_type(xg, jnp.int32)
    keys_ref[...] = jnp.where(bits < 0, bits ^ jnp.int32(0x7FFFFFFF), bits)

    cnt0 = jnp.sum((keys_ref[...] >= 0).astype(jnp.int32), axis=0,
                   keepdims=True)
    t0 = jnp.where(cnt0 >= k, jnp.int32(0), jnp.int32(-(2 ** 31)))

    def body(i, t):
        cand = t | (jnp.int32(1) << (jnp.int32(30) - i))
        cnt = jnp.sum((keys_ref[...] >= cand).astype(jnp.int32), axis=0,
                      keepdims=True)
        return jnp.where(cnt >= k, cand, t)

    t = lax.fori_loop(0, 31, body, t0)
    maskf = (keys_ref[...] >= t).astype(jnp.float32)

    m = jnp.max(xg, axis=-1, keepdims=True)
    e = jnp.exp(xg - m)
    probs = e / jnp.sum(e, axis=-1, keepdims=True)
    out_ref[0] = probs * maskf


def kernel(x, w_gate):
    B, S, D = x.shape
    E = w_gate.shape[0]
    k = max(1, S // E)
    TS = 512
    KS = 1024
    x2 = x.reshape(B * S, D)
    wt = w_gate.T

    xg2 = pl.pallas_call(
        _gemm_body,
        grid=(B * S // TS, D // KS),
        in_specs=[pl.BlockSpec((TS, KS), lambda i, j: (i, j)),
                  pl.BlockSpec((KS, E), lambda i, j: (j, 0))],
        out_specs=pl.BlockSpec((TS, E), lambda i, j: (i, 0)),
        out_shape=jax.ShapeDtypeStruct((B * S, E), jnp.float32),
    )(x2, wt)
    x_gated = xg2.reshape(B, S, E)

    gate = pl.pallas_call(
        functools.partial(_gate_body, k=k),
        grid=(B,),
        in_specs=[pl.BlockSpec((1, S, E), lambda b: (b, 0, 0))],
        out_specs=pl.BlockSpec((1, S, E), lambda b: (b, 0, 0)),
        out_shape=jax.ShapeDtypeStruct((B, S, E), jnp.float32),
        scratch_shapes=[pltpu.VMEM((S, E), jnp.int32)],
    )(x_gated)
    return (gate, x_gated)


# speedup vs baseline: 1.4029x; 1.4029x over previous
import functools

import jax
import jax.numpy as jnp
from jax import lax
from jax.experimental import pallas as pl
from jax.experimental.pallas import tpu as pltpu


def _gemm_body(x_ref, wt_ref, out_ref):
    out_ref[...] = jnp.dot(x_ref[...], wt_ref[...],
                           preferred_element_type=jnp.float32)


def _gate_body(xg_ref, out_ref, keys_ref, *, k):
    xg = xg_ref[0]
    bits = lax.bitcast_convert_type(xg, jnp.int32)
    keys_ref[...] = jnp.where(bits < 0, bits ^ jnp.int32(0x7FFFFFFF), bits)

    cnt0 = jnp.sum((keys_ref[...] >= 0).astype(jnp.int32), axis=0,
                   keepdims=True)
    t0 = jnp.where(cnt0 >= k, jnp.int32(0), jnp.int32(-(2 ** 31)))

    def body(i, t):
        cand = t | (jnp.int32(1) << (jnp.int32(30) - i))
        cnt = jnp.sum((keys_ref[...] >= cand).astype(jnp.int32), axis=0,
                      keepdims=True)
        return jnp.where(cnt >= k, cand, t)

    t = lax.fori_loop(0, 31, body, t0)
    maskf = (keys_ref[...] >= t).astype(jnp.float32)

    m = jnp.max(xg, axis=-1, keepdims=True)
    e = jnp.exp(xg - m)
    probs = e / jnp.sum(e, axis=-1, keepdims=True)
    out_ref[0] = probs * maskf


def kernel(x, w_gate):
    B, S, D = x.shape
    E = w_gate.shape[0]
    k = max(1, S // E)
    TS = 1024
    x2 = x.reshape(B * S, D)
    wt = w_gate.T

    xg2 = pl.pallas_call(
        _gemm_body,
        grid=(B * S // TS,),
        in_specs=[pl.BlockSpec((TS, D), lambda i: (i, 0)),
                  pl.BlockSpec((D, E), lambda i: (0, 0))],
        out_specs=pl.BlockSpec((TS, E), lambda i: (i, 0)),
        out_shape=jax.ShapeDtypeStruct((B * S, E), jnp.float32),
    )(x2, wt)
    x_gated = xg2.reshape(B, S, E)

    gate = pl.pallas_call(
        functools.partial(_gate_body, k=k),
        grid=(B,),
        in_specs=[pl.BlockSpec((1, S, E), lambda b: (b, 0, 0))],
        out_specs=pl.BlockSpec((1, S, E), lambda b: (b, 0, 0)),
        out_shape=jax.ShapeDtypeStruct((B, S, E), jnp.float32),
        scratch_shapes=[pltpu.VMEM((S, E), jnp.int32)],
    )(x_gated)
    return (gate, x_gated)


# speedup vs baseline: 1.4403x; 1.0266x over previous
import functools

import jax
import jax.numpy as jnp
from jax import lax
from jax.experimental import pallas as pl
from jax.experimental.pallas import tpu as pltpu


def _fused_body(x_ref, wt_ref, xg_ref, gate_ref, keys_ref, *, k, spb, ts):
    i = pl.program_id(0)
    xg = jnp.dot(x_ref[...], wt_ref[...], preferred_element_type=jnp.float32)
    xg_ref[...] = xg
    gate_ref[pl.ds((i % spb) * ts, ts), :] = xg

    @pl.when(i % spb == spb - 1)
    def _gate():
        full = gate_ref[...]
        bits = lax.bitcast_convert_type(full, jnp.int32)
        keys_ref[...] = jnp.where(bits < 0, bits ^ jnp.int32(0x7FFFFFFF), bits)

        cnt0 = jnp.sum((keys_ref[...] >= 0).astype(jnp.int32), axis=0,
                       keepdims=True)
        t0 = jnp.where(cnt0 >= k, jnp.int32(0), jnp.int32(-(2 ** 31)))

        def body(j, t):
            cand = t | (jnp.int32(1) << (jnp.int32(30) - j))
            cnt = jnp.sum((keys_ref[...] >= cand).astype(jnp.int32), axis=0,
                          keepdims=True)
            return jnp.where(cnt >= k, cand, t)

        t = lax.fori_loop(0, 31, body, t0)
        maskf = (keys_ref[...] >= t).astype(jnp.float32)

        m = jnp.max(full, axis=-1, keepdims=True)
        e = jnp.exp(full - m)
        probs = e / jnp.sum(e, axis=-1, keepdims=True)
        gate_ref[...] = probs * maskf


def kernel(x, w_gate):
    B, S, D = x.shape
    E = w_gate.shape[0]
    k = max(1, S // E)
    TS = 1024
    SPB = S // TS
    x2 = x.reshape(B * S, D)
    wt = w_gate.T

    xg2, gate2 = pl.pallas_call(
        functools.partial(_fused_body, k=k, spb=SPB, ts=TS),
        grid=(B * S // TS,),
        in_specs=[pl.BlockSpec((TS, D), lambda i: (i, 0)),
                  pl.BlockSpec((D, E), lambda i: (0, 0))],
        out_specs=[pl.BlockSpec((TS, E), lambda i: (i, 0)),
                   pl.BlockSpec((S, E), lambda i: (i // (S // TS), 0))],
        out_shape=[jax.ShapeDtypeStruct((B * S, E), jnp.float32),
                   jax.ShapeDtypeStruct((B * S, E), jnp.float32)],
        scratch_shapes=[pltpu.VMEM((S, E), jnp.int32)],
    )(x2, wt)
    return (gate2.reshape(B, S, E), xg2.reshape(B, S, E))
